# Initial kernel scaffold; baseline (speedup 1.0000x reference)
import jax
import jax.numpy as jnp
from jax.experimental import pallas as pl


def kernel(x, batch_ind, W_mask, b_mask, W_feat, b_feat, W_trans, b_trans):
    raise NotImplementedError("write your pallas kernel here")


# speedup vs baseline: 9.8197x; 9.8197x over previous
import functools

import jax
import jax.numpy as jnp
from jax import lax
from jax.experimental import pallas as pl
from jax.experimental.pallas import tpu as pltpu

_NUM_SEGMENTS = 256


def _pool_body(x_ref, ind_ref, wm_ref, bm_ref, wf_ref, bf_ref, wt_ref,
               bt_ref, out_ref, m_ref, z_ref, p_ref, *, block_rows):
    B = block_rows
    S = _NUM_SEGMENTS
    D = x_ref.shape[1]
    i = pl.program_id(0)
    nb = pl.num_programs(0)

    @pl.when(i == 0)
    def _init():
        m_ref[...] = jnp.full((1, S), -jnp.inf, jnp.float32)
        z_ref[...] = jnp.zeros((1, S), jnp.float32)
        p_ref[...] = jnp.zeros((D, S), jnp.float32)

    xb = x_ref[...]
    ind_col = ind_ref[...].reshape(B, 1)
    valid = ind_col >= 0

    gate = jnp.dot(xb, wm_ref[...], preferred_element_type=jnp.float32)
    gate = gate + bm_ref[0, 0]
    feat = jnp.dot(xb, wf_ref[...], preferred_element_type=jnp.float32)
    feat = feat + bf_ref[...]
    feat = jnp.where(feat >= 0, feat, 0.01 * feat)
    feat = jnp.where(valid, feat, 0.0)

    one_hot = ind_col == lax.broadcasted_iota(jnp.int32, (1, S), 1)
    oh_f = one_hot.astype(jnp.float32)

    m_loc = jnp.max(jnp.where(one_hot, gate, -jnp.inf), axis=0,
                    keepdims=True)
    m_old = m_ref[...]
    m_new = jnp.maximum(m_old, m_loc)
    scale = jnp.where(m_old == -jnp.inf, 0.0, jnp.exp(m_old - m_new))

    m_new_safe = jnp.where(m_new == -jnp.inf, 0.0, m_new)
    m_sel = jnp.sum(oh_f * m_new_safe, axis=1, keepdims=True)
    w = jnp.where(valid, jnp.exp(gate - m_sel), 0.0)
    oh_w = oh_f * w

    z_ref[...] = z_ref[...] * scale + jnp.sum(oh_w, axis=0, keepdims=True)
    p_loc = lax.dot_general(feat, oh_w, (((0,), (0,)), ((), ())),
                            preferred_element_type=jnp.float32)
    p_ref[...] = p_ref[...] * scale + p_loc
    m_ref[...] = m_new

    @pl.when(i == nb - 1)
    def _final():
        pooled_t = p_ref[...] / (z_ref[...] + 1e-16)
        acc = lax.dot_general(pooled_t, wt_ref[...], (((0,), (0,)), ((), ())),
                              preferred_element_type=jnp.float32)
        acc = acc + bt_ref[...]
        out_ref[...] = jnp.where(acc >= 0, acc, 0.01 * acc)


def kernel(x, batch_ind, W_mask, b_mask, W_feat, b_feat, W_trans, b_trans):
    N, D = x.shape
    S = _NUM_SEGMENTS
    B = 2048
    nb = -(-N // B)

    ind = batch_ind.astype(jnp.int32)
    ind = jnp.pad(ind, (0, nb * B - N), constant_values=-1)
    ind3 = ind.reshape(nb, B, 1)

    body = functools.partial(_pool_body, block_rows=B)
    out = pl.pallas_call(
        body,
        grid=(nb,),
        in_specs=[
            pl.BlockSpec((B, D), lambda i: (i, 0)),
            pl.BlockSpec((1, B, 1), lambda i: (i, 0, 0)),
            pl.BlockSpec((D, 1), lambda i: (0, 0)),
            pl.BlockSpec((1, 1), lambda i: (0, 0)),
            pl.BlockSpec((D, D), lambda i: (0, 0)),
            pl.BlockSpec((1, D), lambda i: (0, 0)),
            pl.BlockSpec((D, D), lambda i: (0, 0)),
            pl.BlockSpec((1, D), lambda i: (0, 0)),
        ],
        out_specs=pl.BlockSpec((S, D), lambda i: (0, 0)),
        out_shape=jax.ShapeDtypeStruct((S, D), jnp.float32),
        scratch_shapes=[
            pltpu.VMEM((1, S), jnp.float32),
            pltpu.VMEM((1, S), jnp.float32),
            pltpu.VMEM((D, S), jnp.float32),
        ],
        compiler_params=pltpu.CompilerParams(
            dimension_semantics=("arbitrary",)),
    )(x, ind3, W_mask, b_mask.reshape(1, 1), W_feat, b_feat.reshape(1, D),
      W_trans, b_trans.reshape(1, D))
    return out


# speedup vs baseline: 11.2051x; 1.1411x over previous
import functools

import jax
import jax.numpy as jnp
from jax import lax
from jax.experimental import pallas as pl
from jax.experimental.pallas import tpu as pltpu

_NUM_SEGMENTS = 256


def _pool_body(x_ref, ind_ref, wm_ref, bm_ref, wf_ref, bf_ref, wt_ref,
               bt_ref, out_ref, m_ref, z_ref, p_ref, *, block_rows):
    B = block_rows
    S = _NUM_SEGMENTS
    D = x_ref.shape[1]
    i = pl.program_id(0)
    nb = pl.num_programs(0)

    @pl.when(i == 0)
    def _init():
        m_ref[...] = jnp.full((1, S), -jnp.inf, jnp.float32)
        z_ref[...] = jnp.zeros((1, S), jnp.float32)
        p_ref[...] = jnp.zeros((D, S), jnp.float32)

    ind_col = ind_ref[...].reshape(B, 1)
    valid = ind_col >= 0
    xb = jnp.where(valid, x_ref[...], 0.0)

    gate = jnp.dot(xb, wm_ref[...], preferred_element_type=jnp.float32)
    gate = gate + bm_ref[0, 0]
    feat = jnp.dot(xb, wf_ref[...], preferred_element_type=jnp.float32)
    feat = feat + bf_ref[...]
    feat = jnp.maximum(feat, 0.01 * feat)

    one_hot = ind_col == lax.broadcasted_iota(jnp.int32, (1, S), 1)
    oh_f = one_hot.astype(jnp.float32)

    m_blk = jnp.max(jnp.where(valid, gate, -jnp.inf))
    w = jnp.where(valid, jnp.exp(gate - m_blk), 0.0)
    feat_w = feat * w

    z_loc = lax.dot_general(w, oh_f, (((0,), (0,)), ((), ())),
                            preferred_element_type=jnp.float32)
    p_loc = lax.dot_general(feat_w, oh_f, (((0,), (0,)), ((), ())),
                            preferred_element_type=jnp.float32)

    present = z_loc > 0.0
    m_old = m_ref[...]
    m_new = jnp.where(present, jnp.maximum(m_old, m_blk), m_old)
    scale_old = jnp.where(m_old == -jnp.inf, 0.0, jnp.exp(m_old - m_new))
    scale_loc = jnp.where(present, jnp.exp(m_blk - m_new), 0.0)

    z_ref[...] = z_ref[...] * scale_old + z_loc * scale_loc
    p_ref[...] = p_ref[...] * scale_old + p_loc * scale_loc
    m_ref[...] = m_new

    @pl.when(i == nb - 1)
    def _final():
        pooled_t = p_ref[...] / (z_ref[...] + 1e-16)
        acc = lax.dot_general(pooled_t, wt_ref[...], (((0,), (0,)), ((), ())),
                              preferred_element_type=jnp.float32)
        acc = acc + bt_ref[...]
        out_ref[...] = jnp.where(acc >= 0, acc, 0.01 * acc)


def kernel(x, batch_ind, W_mask, b_mask, W_feat, b_feat, W_trans, b_trans):
    N, D = x.shape
    S = _NUM_SEGMENTS
    B = 2048
    nb = -(-N // B)

    ind = batch_ind.astype(jnp.int32)
    ind = jnp.pad(ind, (0, nb * B - N), constant_values=-1)
    ind3 = ind.reshape(nb, B, 1)

    body = functools.partial(_pool_body, block_rows=B)
    out = pl.pallas_call(
        body,
        grid=(nb,),
        in_specs=[
            pl.BlockSpec((B, D), lambda i: (i, 0)),
            pl.BlockSpec((1, B, 1), lambda i: (i, 0, 0)),
            pl.BlockSpec((D, 1), lambda i: (0, 0)),
            pl.BlockSpec((1, 1), lambda i: (0, 0)),
            pl.BlockSpec((D, D), lambda i: (0, 0)),
            pl.BlockSpec((1, D), lambda i: (0, 0)),
            pl.BlockSpec((D, D), lambda i: (0, 0)),
            pl.BlockSpec((1, D), lambda i: (0, 0)),
        ],
        out_specs=pl.BlockSpec((S, D), lambda i: (0, 0)),
        out_shape=jax.ShapeDtypeStruct((S, D), jnp.float32),
        scratch_shapes=[
            pltpu.VMEM((1, S), jnp.float32),
            pltpu.VMEM((1, S), jnp.float32),
            pltpu.VMEM((D, S), jnp.float32),
        ],
        compiler_params=pltpu.CompilerParams(
            dimension_semantics=("arbitrary",)),
    )(x, ind3, W_mask, b_mask.reshape(1, 1), W_feat, b_feat.reshape(1, D),
      W_trans, b_trans.reshape(1, D))
    return out


# speedup vs baseline: 19.0933x; 1.7040x over previous
import functools

import jax
import jax.numpy as jnp
from jax import lax
from jax.experimental import pallas as pl
from jax.experimental.pallas import tpu as pltpu

_NUM_SEGMENTS = 256


def _pool_body(x_ref, ind_ref, wm_ref, wf_ref, bf_ref, wt_ref,
               bt_ref, out_ref, m_ref, z_ref, p_ref, *, block_rows):
    B = block_rows
    S = _NUM_SEGMENTS
    D = x_ref.shape[1]
    i = pl.program_id(0)
    nb = pl.num_programs(0)

    @pl.when(i == 0)
    def _init():
        m_ref[0] = -jnp.inf
        z_ref[...] = jnp.zeros((S, 1), jnp.float32)
        p_ref[...] = jnp.zeros((S, D), jnp.float32)

    xb = x_ref[...]
    ind_row = ind_ref[...].reshape(1, B)

    gate = jnp.dot(xb, wm_ref[...], preferred_element_type=jnp.float32)
    feat = jnp.dot(xb, wf_ref[...], preferred_element_type=jnp.float32)
    feat = feat + bf_ref[...]
    feat = jnp.maximum(feat, 0.01 * feat)

    gate_row = gate.reshape(1, B)
    m_blk = jnp.max(gate_row)
    w_row = jnp.exp(gate_row - m_blk)

    seg_iota = lax.broadcasted_iota(jnp.int32, (S, 1), 0)
    oh_w = jnp.where(seg_iota == ind_row, w_row, 0.0)

    p_loc = jnp.dot(oh_w, feat, preferred_element_type=jnp.float32)
    z_loc = jnp.dot(oh_w, jnp.ones((B, 1), jnp.float32),
                    preferred_element_type=jnp.float32)

    m_old = m_ref[0]
    m_new = jnp.maximum(m_old, m_blk)
    s_old = jnp.exp(m_old - m_new)
    s_loc = jnp.exp(m_blk - m_new)
    z_ref[...] = z_ref[...] * s_old + z_loc * s_loc
    p_ref[...] = p_ref[...] * s_old + p_loc * s_loc
    m_ref[0] = m_new

    @pl.when(i == nb - 1)
    def _final():
        pooled = p_ref[...] / (z_ref[...] + 1e-16)
        acc = jnp.dot(pooled, wt_ref[...],
                      preferred_element_type=jnp.float32) + bt_ref[...]
        out_ref[...] = jnp.maximum(acc, 0.01 * acc)


def kernel(x, batch_ind, W_mask, b_mask, W_feat, b_feat, W_trans, b_trans):
    del b_mask
    N, D = x.shape
    S = _NUM_SEGMENTS
    B = 2000 if N % 2000 == 0 else 2048
    nb = -(-N // B)

    ind = batch_ind.astype(jnp.int32)
    if nb * B != N:
        x = jnp.pad(x, ((0, nb * B - N), (0, 0)))
        ind = jnp.pad(ind, (0, nb * B - N), constant_values=-1)
    ind3 = ind.reshape(nb, 1, B)

    body = functools.partial(_pool_body, block_rows=B)
    out = pl.pallas_call(
        body,
        grid=(nb,),
        in_specs=[
            pl.BlockSpec((B, D), lambda i: (i, 0)),
            pl.BlockSpec((1, 1, B), lambda i: (i, 0, 0)),
            pl.BlockSpec((D, 1), lambda i: (0, 0)),
            pl.BlockSpec((D, D), lambda i: (0, 0)),
            pl.BlockSpec((1, D), lambda i: (0, 0)),
            pl.BlockSpec((D, D), lambda i: (0, 0)),
            pl.BlockSpec((1, D), lambda i: (0, 0)),
        ],
        out_specs=pl.BlockSpec((S, D), lambda i: (0, 0)),
        out_shape=jax.ShapeDtypeStruct((S, D), jnp.float32),
        scratch_shapes=[
            pltpu.SMEM((1,), jnp.float32),
            pltpu.VMEM((S, 1), jnp.float32),
            pltpu.VMEM((S, D), jnp.float32),
        ],
        compiler_params=pltpu.CompilerParams(
            dimension_semantics=("arbitrary",)),
    )(x, ind3, W_mask, W_feat, b_feat.reshape(1, D),
      W_trans, b_trans.reshape(1, D))
    return out


# speedup vs baseline: 25.3900x; 1.3298x over previous
import functools

import jax
import jax.numpy as jnp
from jax import lax
from jax.experimental import pallas as pl
from jax.experimental.pallas import tpu as pltpu

_NUM_SEGMENTS = 256


def _pool_body(x_ref, ind_ref, wa_ref, bf_ref, wt_ref, bt_ref,
               out_ref, m_ref, p_ref, *, block_rows):
    B = block_rows
    S = _NUM_SEGMENTS
    D = x_ref.shape[1]
    i = pl.program_id(0)
    nb = pl.num_programs(0)

    @pl.when(i == 0)
    def _init():
        m_ref[0] = 0.0
        p_ref[...] = jnp.zeros((S, D + 1), jnp.float32)

    xb = x_ref[...]
    ind_row = ind_ref[...].reshape(1, B)

    raw = jnp.dot(xb, wa_ref[...], preferred_element_type=jnp.float32)
    gate = raw[:, D:D + 1]
    feat = raw[:, :D] + bf_ref[...]
    feat = jnp.maximum(feat, 0.01 * feat)

    m_old = m_ref[0]
    w_col = jnp.exp(gate - m_old)
    feat_aug = jnp.concatenate([feat * w_col, w_col], axis=1)

    seg_iota = lax.broadcasted_iota(jnp.int32, (S, 1), 0)
    oh = jnp.where(seg_iota == ind_row, 1.0, 0.0)

    p_loc = jnp.dot(oh, feat_aug, preferred_element_type=jnp.float32)

    m_new = jnp.maximum(m_old, jnp.max(gate))
    s = jnp.exp(m_old - m_new)
    p_ref[...] = (p_ref[...] + p_loc) * s
    m_ref[0] = m_new

    @pl.when(i == nb - 1)
    def _final():
        acc = p_ref[...]
        pooled = acc[:, :D] / (acc[:, D:D + 1] + 1e-16)
        o = jnp.dot(pooled, wt_ref[...],
                    preferred_element_type=jnp.float32) + bt_ref[...]
        out_ref[...] = jnp.maximum(o, 0.01 * o)


def kernel(x, batch_ind, W_mask, b_mask, W_feat, b_feat, W_trans, b_trans):
    del b_mask
    N, D = x.shape
    S = _NUM_SEGMENTS
    B = 2000 if N % 2000 == 0 else 2048
    nb = -(-N // B)

    ind = batch_ind.astype(jnp.int32)
    if nb * B != N:
        x = jnp.pad(x, ((0, nb * B - N), (0, 0)))
        ind = jnp.pad(ind, (0, nb * B - N), constant_values=-1)
    ind3 = ind.reshape(nb, 1, B)
    W_aug = jnp.concatenate([W_feat, W_mask], axis=1)

    body = functools.partial(_pool_body, block_rows=B)
    out = pl.pallas_call(
        body,
        grid=(nb,),
        in_specs=[
            pl.BlockSpec((B, D), lambda i: (i, 0)),
            pl.BlockSpec((1, 1, B), lambda i: (i, 0, 0)),
            pl.BlockSpec((D, D + 1), lambda i: (0, 0)),
            pl.BlockSpec((1, D), lambda i: (0, 0)),
            pl.BlockSpec((D, D), lambda i: (0, 0)),
            pl.BlockSpec((1, D), lambda i: (0, 0)),
        ],
        out_specs=pl.BlockSpec((S, D), lambda i: (0, 0)),
        out_shape=jax.ShapeDtypeStruct((S, D), jnp.float32),
        scratch_shapes=[
            pltpu.SMEM((1,), jnp.float32),
            pltpu.VMEM((S, D + 1), jnp.float32),
        ],
        compiler_params=pltpu.CompilerParams(
            dimension_semantics=("arbitrary",)),
    )(x, ind3, W_aug, b_feat.reshape(1, D), W_trans, b_trans.reshape(1, D))
    return out
